# Initial kernel scaffold; baseline (speedup 1.0000x reference)
import jax
import jax.numpy as jnp
from jax.experimental import pallas as pl


def kernel(x, edge_index, edge_values, weight, self_weight, bn_gamma, bn_beta):
    raise NotImplementedError("write your pallas kernel here")


# speedup vs baseline: 3.9817x; 3.9817x over previous
import functools

import jax
import jax.numpy as jnp
from jax import lax
from jax.experimental import pallas as pl
from jax.experimental.pallas import tpu as pltpu
from jax.experimental.pallas import tpu_sc as plsc

N = 10000
E = 320000
D = 128
T = 5
BETA = 0.99
EPS_ORTHO = 1e-05
EPS_BN = 1e-05

NC = 2
NS = 16
NW = NC * NS
EPW = E // NW
CHUNK = 80
NCHUNK = EPW // CHUNK
RPS = N // NS
ZROWS = 125


def _sc_scatter_body(support_hbm, row_hbm, col_hbm, val_hbm, out_hbm,
                     col_v, row_v, val_v, rows_v, zbuf, acc, sem):
  c = lax.axis_index("c")
  s = lax.axis_index("s")
  wid = c * NS + s

  if True:
    zeros16 = jnp.zeros((16,), jnp.float32)

    def zrow(r, carry):
      for g in range(D // 16):
        zbuf[r, pl.ds(g * 16, 16)] = zeros16
      return carry

    lax.fori_loop(0, ZROWS, zrow, 0)
    rbase = s * RPS
    for k in range(RPS // ZROWS):
      pltpu.sync_copy(zbuf, acc.at[pl.ds(rbase + k * ZROWS, ZROWS)])
    plsc.subcore_barrier()

    ebase = wid * EPW

    def chunk_body(ci, carry):
      off = ebase + ci * CHUNK
      pltpu.sync_copy(col_hbm.at[pl.ds(off, CHUNK)], col_v)
      pltpu.sync_copy(row_hbm.at[pl.ds(off, CHUNK)], row_v)
      pltpu.sync_copy(val_hbm.at[pl.ds(off, CHUNK)], val_v)
      pltpu.async_copy(support_hbm.at[col_v], rows_v, sem).wait()

      def group_body(gi, ecarry):
        val16 = val_v[pl.ds(gi * 16, 16)]
        for i in range(16):
          vb = lax.gather(
              val16, jnp.full((16, 1), i, jnp.int32),
              lax.GatherDimensionNumbers(
                  offset_dims=(), collapsed_slice_dims=(0,),
                  start_index_map=(0,)),
              slice_sizes=(1,),
              mode=lax.GatherScatterMode.PROMISE_IN_BOUNDS)
          e = gi * 16 + i
          for g in range(D // 16):
            rows_v[e, pl.ds(g * 16, 16)] = rows_v[e, pl.ds(g * 16, 16)] * vb
        return ecarry

      lax.fori_loop(0, CHUNK // 16, group_body, 0)
      pltpu.sync_copy(rows_v, acc.at[row_v], add=True)
      return carry

    lax.fori_loop(0, NCHUNK, chunk_body, 0)
    plsc.subcore_barrier()

    pltpu.sync_copy(acc.at[pl.ds(rbase, RPS)], out_hbm.at[c, s])


@functools.partial(jax.jit, static_argnames=())
def _sc_scatter(support, row, col, val):
  mesh = plsc.VectorSubcoreMesh(core_axis_name="c", subcore_axis_name="s")
  return pl.kernel(
      _sc_scatter_body,
      out_type=jax.ShapeDtypeStruct((NC, NS, RPS, D), jnp.float32),
      mesh=mesh,
      scratch_types=[
          pltpu.VMEM((CHUNK,), jnp.int32),
          pltpu.VMEM((CHUNK,), jnp.int32),
          pltpu.VMEM((CHUNK,), jnp.float32),
          pltpu.VMEM((CHUNK, D), jnp.float32),
          pltpu.VMEM((ZROWS, D), jnp.float32),
          pltpu.VMEM_SHARED((N, D), jnp.float32),
          pltpu.SemaphoreType.DMA,
      ],
  )(support, row, col, val)


def _eye(n, dtype):
  i = lax.broadcasted_iota(jnp.int32, (n, n), 0)
  j = lax.broadcasted_iota(jnp.int32, (n, n), 1)
  return jnp.where(i == j, 1.0, 0.0).astype(dtype)


def _dot(a, b):
  return jax.lax.dot(a, b, precision=jax.lax.Precision.DEFAULT)


def _tc_ortho_support_body(x_ref, w_ref, out_ref):
  w = w_ref[...]
  eye = _eye(D, jnp.float32)
  we = BETA * w + (1.0 - BETA) * eye
  zc = we - jnp.mean(we, axis=1, keepdims=True)
  s = _dot(zc, zc.T)
  s = s + EPS_ORTHO * eye
  norm = jnp.sqrt(jnp.sum(s * s))
  s = s / norm
  b = eye
  for _ in range(T):
    b3 = _dot(_dot(b, b), b)
    b = 1.5 * b - 0.5 * _dot(b3, s)
  t = _dot(b, zc) / jnp.sqrt(norm)
  out_ref[...] = _dot(x_ref[...], t)


def _tc_finish_body(x_ref, sw_ref, a0_ref, a1_ref, g_ref, b_ref, out_ref):
  o = a0_ref[...] + a1_ref[...] + _dot(x_ref[...], sw_ref[...])
  mean = jnp.mean(o, axis=0, keepdims=True)
  cen = o - mean
  var = jnp.mean(cen * cen, axis=0, keepdims=True)
  out_ref[...] = cen * (g_ref[...] * jax.lax.rsqrt(var + EPS_BN)) + b_ref[...]


def kernel(x, edge_index, edge_values, weight, self_weight, bn_gamma, bn_beta):
  support = pl.pallas_call(
      _tc_ortho_support_body,
      out_shape=jax.ShapeDtypeStruct((N, D), jnp.float32),
  )(x, weight)

  parts = _sc_scatter(support, edge_index[0], edge_index[1], edge_values)
  parts = parts.reshape(NC, N, D)

  out = pl.pallas_call(
      _tc_finish_body,
      out_shape=jax.ShapeDtypeStruct((N, D), jnp.float32),
  )(x, self_weight, parts[0], parts[1],
    bn_gamma.reshape(1, D), bn_beta.reshape(1, D))
  return out
